# baseline (speedup 1.0000x reference)
import functools

import jax
import jax.numpy as jnp
from jax import lax
from jax.experimental import pallas as pl
from jax.experimental.pallas import tpu as pltpu
from jax.experimental.pallas import tpu_sc as plsc

_E = 64
_TOPK = 2
_D = 768
_INTER = 256
_BLK = 256
_NB = 128
_GATE_TB = 512
_TB = 512
_DH = _D // 2


def _pack_rows(xb):
    lo = lax.bitcast_convert_type(xb[:, :_DH], jnp.uint16).astype(jnp.uint32)
    hi = lax.bitcast_convert_type(xb[:, _DH:], jnp.uint16).astype(jnp.uint32)
    return lax.bitcast_convert_type(lo | (hi << 16), jnp.int32)


def _unpack_rows(u):
    u = lax.bitcast_convert_type(u, jnp.uint32)
    lo = lax.bitcast_convert_type((u & 0xFFFF).astype(jnp.uint16), jnp.bfloat16)
    hi = lax.bitcast_convert_type((u >> 16).astype(jnp.uint16), jnp.bfloat16)
    return jnp.concatenate([lo, hi], axis=1)


def _gate_body(x_ref, gw_ref, tril_ref, idx_ref, w_ref, rank_ref, cnt_ref,
               xbf_ref, carry_ref):
    b = pl.program_id(0)
    x = x_ref[...]
    xbf_ref[...] = _pack_rows(x.astype(jnp.bfloat16))
    g = gw_ref[...]
    logits = lax.dot_general(x, g, (((1,), (1,)), ((), ())),
                             preferred_element_type=jnp.float32)
    lanes = lax.broadcasted_iota(jnp.int32, logits.shape, 1)
    m1 = jnp.max(logits, axis=1, keepdims=True)
    i1 = jnp.min(jnp.where(logits == m1, lanes, _E), axis=1, keepdims=True)
    masked = jnp.where(lanes == i1, -jnp.inf, logits)
    m2 = jnp.max(masked, axis=1, keepdims=True)
    i2 = jnp.min(jnp.where(masked == m2, lanes, _E), axis=1, keepdims=True)
    w1 = 1.0 / (1.0 + jnp.exp(m2 - m1))
    w2 = 1.0 - w1
    idx_ref[...] = jnp.concatenate([i1, i2], axis=1)
    w_ref[...] = jnp.concatenate([w1, w2], axis=1)

    @pl.when(b == 0)
    def _():
        carry_ref[...] = jnp.zeros((1, _E), jnp.float32)

    carry = carry_ref[...]
    oh1 = (lanes == i1).astype(jnp.float32)
    oh2 = (lanes == i2).astype(jnp.float32)
    s = oh1 + oh2
    pex = lax.dot_general(tril_ref[...], s, (((1,), (0,)), ((), ())),
                          preferred_element_type=jnp.float32)
    r0 = jnp.sum((pex + carry) * oh1, axis=1, keepdims=True)
    r1 = jnp.sum((pex + carry) * oh2, axis=1, keepdims=True)
    rank_ref[...] = jnp.concatenate([r0, r1], axis=1).astype(jnp.int32)
    new_carry = carry + jnp.sum(s, axis=0, keepdims=True)
    carry_ref[...] = new_carry
    cnt_ref[...] = new_carry.astype(jnp.int32)


def _gate(xf, gate_w, tril):
    n = xf.shape[0]
    grid = (n // _GATE_TB,)
    return pl.pallas_call(
        _gate_body,
        grid=grid,
        in_specs=[
            pl.BlockSpec((_GATE_TB, _D), lambda b: (b, 0)),
            pl.BlockSpec((_E, _D), lambda b: (0, 0)),
            pl.BlockSpec((_GATE_TB, _GATE_TB), lambda b: (0, 0)),
        ],
        out_specs=[
            pl.BlockSpec((_GATE_TB, _TOPK), lambda b: (b, 0)),
            pl.BlockSpec((_GATE_TB, _TOPK), lambda b: (b, 0)),
            pl.BlockSpec((_GATE_TB, _TOPK), lambda b: (b, 0)),
            pl.BlockSpec((1, _E), lambda b: (0, 0)),
            pl.BlockSpec((_GATE_TB, _DH), lambda b: (b, 0)),
        ],
        out_shape=[
            jax.ShapeDtypeStruct((n, _TOPK), jnp.int32),
            jax.ShapeDtypeStruct((n, _TOPK), jnp.float32),
            jax.ShapeDtypeStruct((n, _TOPK), jnp.int32),
            jax.ShapeDtypeStruct((1, _E), jnp.int32),
            jax.ShapeDtypeStruct((n, _DH), jnp.int32),
        ],
        scratch_shapes=[pltpu.VMEM((1, _E), jnp.float32)],
    )(xf, gate_w, tril)


def _pp_body(idx_ref, rank_ref, po_ref, pp_ref):
    idx = idx_ref[...]
    lanes = lax.broadcasted_iota(jnp.int32, (_GATE_TB, _E), 1)
    po = po_ref[...].astype(jnp.float32)
    oh1 = (lanes == idx[:, 0:1]).astype(jnp.float32)
    oh2 = (lanes == idx[:, 1:2]).astype(jnp.float32)
    p0 = jnp.sum(oh1 * po, axis=1, keepdims=True)
    p1 = jnp.sum(oh2 * po, axis=1, keepdims=True)
    pp_ref[...] = (jnp.concatenate([p0, p1], axis=1).astype(jnp.int32)
                   + rank_ref[...])


def _pp_compute(top_idx, rank2, pad_offs):
    n = top_idx.shape[0]
    return pl.pallas_call(
        _pp_body,
        grid=(n // _GATE_TB,),
        in_specs=[
            pl.BlockSpec((_GATE_TB, _TOPK), lambda b: (b, 0)),
            pl.BlockSpec((_GATE_TB, _TOPK), lambda b: (b, 0)),
            pl.BlockSpec((1, _E), lambda b: (0, 0)),
        ],
        out_specs=pl.BlockSpec((_GATE_TB, _TOPK), lambda b: (b, 0)),
        out_shape=jax.ShapeDtypeStruct((n, _TOPK), jnp.int32),
    )(top_idx, rank2, pad_offs)


_CH = 128


def _sc_dispatch(table, pp, out_rows):
    b, = pp.shape
    n, d = table.shape
    nw = 32
    per_w = b // nw
    n_ch = per_w // _CH
    mesh = plsc.VectorSubcoreMesh(core_axis_name="c", subcore_axis_name="s")

    @functools.partial(
        pl.kernel,
        mesh=mesh,
        out_type=jax.ShapeDtypeStruct((out_rows, d), table.dtype),
        scratch_types=[
            pltpu.VMEM((_CH,), jnp.int32),
            pltpu.VMEM((_CH, d), table.dtype),
            pltpu.SemaphoreType.DMA,
        ],
    )
    def k(table_hbm, pp_hbm, out_hbm, pp_v, rows_v, sem):
        wid = lax.axis_index("s") * 2 + lax.axis_index("c")

        def body(c, carry):
            base = wid * per_w + c * _CH
            tok_base = base - (base // n) * n
            pltpu.sync_copy(pp_hbm.at[pl.ds(base, _CH)], pp_v)
            pltpu.sync_copy(table_hbm.at[pl.ds(tok_base, _CH)], rows_v)
            pltpu.async_copy(rows_v, out_hbm.at[pp_v], sem).wait()
            return carry

        lax.fori_loop(0, n_ch, body, 0)

    return k(table, pp)


def _sc_undispatch(table, pp):
    b, = pp.shape
    d = table.shape[1]
    nw = 32
    per_w = b // nw
    n_ch = per_w // _CH
    mesh = plsc.VectorSubcoreMesh(core_axis_name="c", subcore_axis_name="s")

    @functools.partial(
        pl.kernel,
        mesh=mesh,
        out_type=jax.ShapeDtypeStruct((b, d), table.dtype),
        scratch_types=[
            pltpu.VMEM((_CH,), jnp.int32),
            pltpu.VMEM((_CH, d), table.dtype),
            pltpu.SemaphoreType.DMA,
        ],
    )
    def k(table_hbm, pp_hbm, out_hbm, pp_v, rows_v, sem):
        wid = lax.axis_index("s") * 2 + lax.axis_index("c")

        def body(c, carry):
            base = wid * per_w + c * _CH
            pltpu.sync_copy(pp_hbm.at[pl.ds(base, _CH)], pp_v)
            pltpu.async_copy(table_hbm.at[pp_v], rows_v, sem).wait()
            pltpu.sync_copy(rows_v, out_hbm.at[pl.ds(base, _CH)])
            return carry

        lax.fori_loop(0, n_ch, body, 0)

    return k(table, pp)


def _expert_mlp(xi, blen, w1_ref, b1_ref, w2_ref, b2_ref):
    x = _unpack_rows(xi)
    rows = lax.broadcasted_iota(jnp.int32, (_BLK, 1), 0)
    xb = jnp.where(rows < blen, x, jnp.bfloat16(0))
    w1 = w1_ref[0].astype(jnp.bfloat16)
    h = lax.dot_general(xb, w1, (((1,), (1,)), ((), ())),
                        preferred_element_type=jnp.float32)
    h = h + b1_ref[0]
    h = 0.5 * h * (1.0 + lax.erf(h * 0.7071067811865476))
    hb = h.astype(jnp.bfloat16)
    w2 = w2_ref[0].astype(jnp.bfloat16)
    y = lax.dot_general(hb, w2, (((1,), (1,)), ((), ())),
                        preferred_element_type=jnp.float32)
    return _pack_rows((y + b2_ref[0]).astype(jnp.bfloat16))


def _ggemm_body(meta_ref, x_ref, w1a_ref, b1a_ref, w2a_ref, b2a_ref,
                w1b_ref, b1b_ref, w2b_ref, b2b_ref, out_ref):
    b = pl.program_id(0)
    nh = _NB // 2
    xx = x_ref[...]
    out_ref[0, 0] = _expert_mlp(xx[0, 0], meta_ref[_NB + b],
                                w1a_ref, b1a_ref, w2a_ref, b2a_ref)
    out_ref[1, 0] = _expert_mlp(xx[1, 0], meta_ref[_NB + nh + b],
                                w1b_ref, b1b_ref, w2b_ref, b2b_ref)


def _grouped_gemm(xg, fc1_w, fc1_b, fc2_w, fc2_b, meta):
    nh = _NB // 2
    grid_spec = pltpu.PrefetchScalarGridSpec(
        num_scalar_prefetch=1,
        grid=(nh,),
        in_specs=[
            pl.BlockSpec((2, 1, _BLK, _DH), lambda b, m: (0, b, 0, 0)),
            pl.BlockSpec((1, _INTER, _D), lambda b, m: (m[b], 0, 0)),
            pl.BlockSpec((1, 1, _INTER), lambda b, m: (m[b], 0, 0)),
            pl.BlockSpec((1, _D, _INTER), lambda b, m: (m[b], 0, 0)),
            pl.BlockSpec((1, 1, _D), lambda b, m: (m[b], 0, 0)),
            pl.BlockSpec((1, _INTER, _D), lambda b, m, _h=nh: (m[_h + b], 0, 0)),
            pl.BlockSpec((1, 1, _INTER), lambda b, m, _h=nh: (m[_h + b], 0, 0)),
            pl.BlockSpec((1, _D, _INTER), lambda b, m, _h=nh: (m[_h + b], 0, 0)),
            pl.BlockSpec((1, 1, _D), lambda b, m, _h=nh: (m[_h + b], 0, 0)),
        ],
        out_specs=pl.BlockSpec((2, 1, _BLK, _DH), lambda b, m: (0, b, 0, 0)),
    )
    xg4 = xg.reshape(2, nh, _BLK, _DH)
    out = pl.pallas_call(
        _ggemm_body,
        grid_spec=grid_spec,
        out_shape=jax.ShapeDtypeStruct((2, nh, _BLK, _DH), jnp.int32),
    )(meta, xg4, fc1_w, fc1_b, fc2_w, fc2_b, fc1_w, fc1_b, fc2_w, fc2_b)
    return out.reshape(_NB * _BLK, _DH)


def _shared_body(x_ref, w1_ref, b1_ref, w2_ref, b2_ref, s_ref):
    x = x_ref[...]
    h = lax.dot_general(x, w1_ref[...], (((1,), (1,)), ((), ())),
                        preferred_element_type=jnp.float32)
    h = h + b1_ref[...]
    h = 0.5 * h * (1.0 + lax.erf(h * 0.7071067811865476))
    s = lax.dot_general(h, w2_ref[...], (((1,), (1,)), ((), ())),
                        preferred_element_type=jnp.float32)
    s_ref[...] = s + b2_ref[...]


def _shared_mlp(xf, sfc1_w, sfc1_b, sfc2_w, sfc2_b):
    n = xf.shape[0]
    return pl.pallas_call(
        _shared_body,
        grid=(n // _TB,),
        in_specs=[
            pl.BlockSpec((_TB, _D), lambda b: (b, 0)),
            pl.BlockSpec((_INTER, _D), lambda b: (0, 0)),
            pl.BlockSpec((1, _INTER), lambda b: (0, 0)),
            pl.BlockSpec((_D, _INTER), lambda b: (0, 0)),
            pl.BlockSpec((1, _D), lambda b: (0, 0)),
        ],
        out_specs=pl.BlockSpec((_TB, _D), lambda b: (b, 0)),
        out_shape=jax.ShapeDtypeStruct((n, _D), jnp.float32),
    )(xf, sfc1_w, sfc1_b, sfc2_w, sfc2_b)


def _combine_body(s_ref, y0_ref, y1_ref, tw_ref, out_ref):
    tw = tw_ref[...]
    y0 = _unpack_rows(y0_ref[...]).astype(jnp.float32)
    y1 = _unpack_rows(y1_ref[...]).astype(jnp.float32)
    out_ref[...] = s_ref[...] + tw[:, 0:1] * y0 + tw[:, 1:2] * y1


def _combine(s, ypair, top_w):
    n = s.shape[0]
    nblk = n // _TB
    return pl.pallas_call(
        _combine_body,
        grid=(nblk,),
        in_specs=[
            pl.BlockSpec((_TB, _D), lambda b: (b, 0)),
            pl.BlockSpec((_TB, _DH), lambda b: (b, 0)),
            pl.BlockSpec((_TB, _DH), lambda b, _nb=nblk: (b + _nb, 0)),
            pl.BlockSpec((_TB, _TOPK), lambda b: (b, 0)),
        ],
        out_specs=pl.BlockSpec((_TB, _D), lambda b: (b, 0)),
        out_shape=jax.ShapeDtypeStruct((n, _D), jnp.float32),
    )(s, ypair, ypair, top_w)


def kernel(x, gate_w, fc1_w, fc1_b, fc2_w, fc2_b, sfc1_w, sfc1_b, sfc2_w, sfc2_b):
    bb, hh, ww, dm = x.shape
    n = bb * hh * ww
    p = n * _TOPK
    xf = x.reshape(n, dm)

    row = jnp.arange(_GATE_TB, dtype=jnp.int32)
    tril = (row[:, None] > row[None, :]).astype(jnp.float32)
    top_idx, top_w, rank2, counts2, xbf = _gate(xf, gate_w, tril)

    counts = counts2.reshape(-1)
    blocks_e = (counts + _BLK - 1) // _BLK
    first_blk = jnp.cumsum(blocks_e) - blocks_e
    pad_offs = _BLK * first_blk
    block_expert = jnp.repeat(jnp.arange(_E, dtype=jnp.int32), blocks_e,
                              total_repeat_length=_NB)
    block_ord = jnp.arange(_NB, dtype=jnp.int32) - first_blk[block_expert]
    block_len = jnp.clip(counts[block_expert] - block_ord * _BLK, 0, _BLK)
    meta = jnp.concatenate([block_expert, block_len]).astype(jnp.int32)
    pp2 = _pp_compute(top_idx, rank2, pad_offs.reshape(1, _E))
    pp = pp2.T.reshape(-1)

    xg = _sc_dispatch(xbf, pp, _NB * _BLK)

    yg = _grouped_gemm(xg, fc1_w, fc1_b.reshape(_E, 1, _INTER),
                       fc2_w, fc2_b.reshape(_E, 1, _D), meta)

    ypair = _sc_undispatch(yg, pp)

    s = _shared_mlp(xf, sfc1_w, sfc1_b.reshape(1, _INTER),
                    sfc2_w, sfc2_b.reshape(1, _D))
    out = _combine(s, ypair, top_w)
    return out.reshape(bb, hh, ww, dm)


# speedup vs baseline: 1.0340x; 1.0340x over previous
import functools

import jax
import jax.numpy as jnp
from jax import lax
from jax.experimental import pallas as pl
from jax.experimental.pallas import tpu as pltpu
from jax.experimental.pallas import tpu_sc as plsc

_E = 64
_TOPK = 2
_D = 768
_INTER = 256
_BLK = 256
_NB = 128
_GATE_TB = 512
_TB = 512
_DH = _D // 2


def _pack_rows(xb):
    lo = lax.bitcast_convert_type(xb[:, :_DH], jnp.uint16).astype(jnp.uint32)
    hi = lax.bitcast_convert_type(xb[:, _DH:], jnp.uint16).astype(jnp.uint32)
    return lax.bitcast_convert_type(lo | (hi << 16), jnp.int32)


def _unpack_rows(u):
    u = lax.bitcast_convert_type(u, jnp.uint32)
    lo = lax.bitcast_convert_type((u & 0xFFFF).astype(jnp.uint16), jnp.bfloat16)
    hi = lax.bitcast_convert_type((u >> 16).astype(jnp.uint16), jnp.bfloat16)
    return jnp.concatenate([lo, hi], axis=1)


def _gate_body(x_ref, gw_ref, tril_ref, idx_ref, w_ref, rank_ref, cnt_ref,
               xbf_ref, carry_ref):
    b = pl.program_id(0)
    x = x_ref[...]
    xbf_ref[...] = _pack_rows(x.astype(jnp.bfloat16))
    g = gw_ref[...]
    logits = lax.dot_general(x, g, (((1,), (1,)), ((), ())),
                             preferred_element_type=jnp.float32)
    lanes = lax.broadcasted_iota(jnp.int32, logits.shape, 1)
    m1 = jnp.max(logits, axis=1, keepdims=True)
    i1 = jnp.min(jnp.where(logits == m1, lanes, _E), axis=1, keepdims=True)
    masked = jnp.where(lanes == i1, -jnp.inf, logits)
    m2 = jnp.max(masked, axis=1, keepdims=True)
    i2 = jnp.min(jnp.where(masked == m2, lanes, _E), axis=1, keepdims=True)
    w1 = 1.0 / (1.0 + jnp.exp(m2 - m1))
    w2 = 1.0 - w1
    idx_ref[...] = jnp.concatenate([i1, i2], axis=1)
    w_ref[...] = jnp.concatenate([w1, w2], axis=1)

    @pl.when(b == 0)
    def _():
        carry_ref[...] = jnp.zeros((1, _E), jnp.float32)

    carry = carry_ref[...]
    oh1 = (lanes == i1).astype(jnp.float32)
    oh2 = (lanes == i2).astype(jnp.float32)
    s = oh1 + oh2
    pex = lax.dot_general(tril_ref[...], s, (((1,), (0,)), ((), ())),
                          preferred_element_type=jnp.float32)
    r0 = jnp.sum((pex + carry) * oh1, axis=1, keepdims=True)
    r1 = jnp.sum((pex + carry) * oh2, axis=1, keepdims=True)
    rank_ref[...] = jnp.concatenate([r0, r1], axis=1).astype(jnp.int32)
    new_carry = carry + jnp.sum(s, axis=0, keepdims=True)
    carry_ref[...] = new_carry
    cnt_ref[...] = new_carry.astype(jnp.int32)


def _gate(xf, gate_w, tril):
    n = xf.shape[0]
    grid = (n // _GATE_TB,)
    return pl.pallas_call(
        _gate_body,
        grid=grid,
        in_specs=[
            pl.BlockSpec((_GATE_TB, _D), lambda b: (b, 0)),
            pl.BlockSpec((_E, _D), lambda b: (0, 0)),
            pl.BlockSpec((_GATE_TB, _GATE_TB), lambda b: (0, 0)),
        ],
        out_specs=[
            pl.BlockSpec((_GATE_TB, _TOPK), lambda b: (b, 0)),
            pl.BlockSpec((_GATE_TB, _TOPK), lambda b: (b, 0)),
            pl.BlockSpec((_GATE_TB, _TOPK), lambda b: (b, 0)),
            pl.BlockSpec((1, _E), lambda b: (0, 0)),
            pl.BlockSpec((_GATE_TB, _DH), lambda b: (b, 0)),
        ],
        out_shape=[
            jax.ShapeDtypeStruct((n, _TOPK), jnp.int32),
            jax.ShapeDtypeStruct((n, _TOPK), jnp.float32),
            jax.ShapeDtypeStruct((n, _TOPK), jnp.int32),
            jax.ShapeDtypeStruct((1, _E), jnp.int32),
            jax.ShapeDtypeStruct((n, _DH), jnp.int32),
        ],
        scratch_shapes=[pltpu.VMEM((1, _E), jnp.float32)],
    )(xf, gate_w, tril)


def _pp_body(idx_ref, rank_ref, po_ref, pp_ref):
    idx = idx_ref[...]
    lanes = lax.broadcasted_iota(jnp.int32, (_GATE_TB, _E), 1)
    po = po_ref[...].astype(jnp.float32)
    oh1 = (lanes == idx[:, 0:1]).astype(jnp.float32)
    oh2 = (lanes == idx[:, 1:2]).astype(jnp.float32)
    p0 = jnp.sum(oh1 * po, axis=1, keepdims=True)
    p1 = jnp.sum(oh2 * po, axis=1, keepdims=True)
    pp_ref[...] = (jnp.concatenate([p0, p1], axis=1).astype(jnp.int32)
                   + rank_ref[...])


def _pp_compute(top_idx, rank2, pad_offs):
    n = top_idx.shape[0]
    return pl.pallas_call(
        _pp_body,
        grid=(n // _GATE_TB,),
        in_specs=[
            pl.BlockSpec((_GATE_TB, _TOPK), lambda b: (b, 0)),
            pl.BlockSpec((_GATE_TB, _TOPK), lambda b: (b, 0)),
            pl.BlockSpec((1, _E), lambda b: (0, 0)),
        ],
        out_specs=pl.BlockSpec((_GATE_TB, _TOPK), lambda b: (b, 0)),
        out_shape=jax.ShapeDtypeStruct((n, _TOPK), jnp.int32),
    )(top_idx, rank2, pad_offs)


_CH = 128


def _sc_dispatch(table, pp, out_rows):
    b, = pp.shape
    n, d = table.shape
    nw = 32
    per_w = b // nw
    n_ch = per_w // _CH
    mesh = plsc.VectorSubcoreMesh(core_axis_name="c", subcore_axis_name="s")

    @functools.partial(
        pl.kernel,
        mesh=mesh,
        out_type=jax.ShapeDtypeStruct((out_rows, d), table.dtype),
        scratch_types=[
            pltpu.VMEM((_CH,), jnp.int32),
            pltpu.VMEM((_CH, d), table.dtype),
            pltpu.SemaphoreType.DMA,
        ],
    )
    def k(table_hbm, pp_hbm, out_hbm, pp_v, rows_v, sem):
        wid = lax.axis_index("s") * 2 + lax.axis_index("c")

        def body(c, carry):
            base = wid * per_w + c * _CH
            tok_base = base - (base // n) * n
            pltpu.sync_copy(pp_hbm.at[pl.ds(base, _CH)], pp_v)
            pltpu.sync_copy(table_hbm.at[pl.ds(tok_base, _CH)], rows_v)
            pltpu.async_copy(rows_v, out_hbm.at[pp_v], sem).wait()
            return carry

        lax.fori_loop(0, n_ch, body, 0)

    return k(table, pp)


def _sc_undispatch(table, pp):
    b, = pp.shape
    d = table.shape[1]
    nw = 32
    per_w = b // nw
    n_ch = per_w // _CH
    mesh = plsc.VectorSubcoreMesh(core_axis_name="c", subcore_axis_name="s")

    @functools.partial(
        pl.kernel,
        mesh=mesh,
        out_type=jax.ShapeDtypeStruct((b, d), table.dtype),
        scratch_types=[
            pltpu.VMEM((_CH,), jnp.int32),
            pltpu.VMEM((_CH, d), table.dtype),
            pltpu.SemaphoreType.DMA,
        ],
    )
    def k(table_hbm, pp_hbm, out_hbm, pp_v, rows_v, sem):
        wid = lax.axis_index("s") * 2 + lax.axis_index("c")

        def body(c, carry):
            base = wid * per_w + c * _CH
            pltpu.sync_copy(pp_hbm.at[pl.ds(base, _CH)], pp_v)
            pltpu.async_copy(table_hbm.at[pp_v], rows_v, sem).wait()
            pltpu.sync_copy(rows_v, out_hbm.at[pl.ds(base, _CH)])
            return carry

        lax.fori_loop(0, n_ch, body, 0)

    return k(table, pp)


def _expert_mlp(xi, blen, w1_ref, b1_ref, w2_ref, b2_ref):
    x = _unpack_rows(xi)
    rows = lax.broadcasted_iota(jnp.int32, (_BLK, 1), 0)
    xb = jnp.where(rows < blen, x, jnp.bfloat16(0))
    w1 = w1_ref[0].astype(jnp.bfloat16)
    h = lax.dot_general(xb, w1, (((1,), (1,)), ((), ())),
                        preferred_element_type=jnp.float32)
    h = h + b1_ref[0]
    h = 0.5 * h * (1.0 + lax.erf(h * 0.7071067811865476))
    hb = h.astype(jnp.bfloat16)
    w2 = w2_ref[0].astype(jnp.bfloat16)
    y = lax.dot_general(hb, w2, (((1,), (1,)), ((), ())),
                        preferred_element_type=jnp.float32)
    return _pack_rows((y + b2_ref[0]).astype(jnp.bfloat16))


def _ggemm_body(meta_ref, x_ref, w1a_ref, b1a_ref, w2a_ref, b2a_ref,
                w1b_ref, b1b_ref, w2b_ref, b2b_ref, out_ref):
    b = pl.program_id(0)
    nh = _NB // 2
    xx = x_ref[...]
    out_ref[0, 0] = _expert_mlp(xx[0, 0], meta_ref[_NB + b],
                                w1a_ref, b1a_ref, w2a_ref, b2a_ref)
    out_ref[1, 0] = _expert_mlp(xx[1, 0], meta_ref[_NB + nh + b],
                                w1b_ref, b1b_ref, w2b_ref, b2b_ref)


def _grouped_gemm(xg, fc1_w, fc1_b, fc2_w, fc2_b, meta):
    nh = _NB // 2
    grid_spec = pltpu.PrefetchScalarGridSpec(
        num_scalar_prefetch=1,
        grid=(nh,),
        in_specs=[
            pl.BlockSpec((2, 1, _BLK, _DH), lambda b, m: (0, b, 0, 0)),
            pl.BlockSpec((1, _INTER, _D), lambda b, m: (m[b], 0, 0)),
            pl.BlockSpec((1, 1, _INTER), lambda b, m: (m[b], 0, 0)),
            pl.BlockSpec((1, _D, _INTER), lambda b, m: (m[b], 0, 0)),
            pl.BlockSpec((1, 1, _D), lambda b, m: (m[b], 0, 0)),
            pl.BlockSpec((1, _INTER, _D), lambda b, m, _h=nh: (m[_h + b], 0, 0)),
            pl.BlockSpec((1, 1, _INTER), lambda b, m, _h=nh: (m[_h + b], 0, 0)),
            pl.BlockSpec((1, _D, _INTER), lambda b, m, _h=nh: (m[_h + b], 0, 0)),
            pl.BlockSpec((1, 1, _D), lambda b, m, _h=nh: (m[_h + b], 0, 0)),
        ],
        out_specs=pl.BlockSpec((2, 1, _BLK, _DH), lambda b, m: (0, b, 0, 0)),
    )
    xg4 = xg.reshape(2, nh, _BLK, _DH)
    out = pl.pallas_call(
        _ggemm_body,
        grid_spec=grid_spec,
        out_shape=jax.ShapeDtypeStruct((2, nh, _BLK, _DH), jnp.int32),
    )(meta, xg4, fc1_w, fc1_b, fc2_w, fc2_b, fc1_w, fc1_b, fc2_w, fc2_b)
    return out.reshape(_NB * _BLK, _DH)


def _combine_body(x_ref, w1_ref, b1_ref, w2_ref, b2_ref, y0_ref, y1_ref,
                  tw_ref, out_ref):
    x = x_ref[...]
    h = lax.dot_general(x, w1_ref[...], (((1,), (1,)), ((), ())),
                        preferred_element_type=jnp.float32)
    h = h + b1_ref[...]
    h = 0.5 * h * (1.0 + lax.erf(h * 0.7071067811865476))
    s = lax.dot_general(h, w2_ref[...], (((1,), (1,)), ((), ())),
                        preferred_element_type=jnp.float32)
    s = s + b2_ref[...]
    tw = tw_ref[...]
    y0 = _unpack_rows(y0_ref[...]).astype(jnp.float32)
    y1 = _unpack_rows(y1_ref[...]).astype(jnp.float32)
    out_ref[...] = s + tw[:, 0:1] * y0 + tw[:, 1:2] * y1


def _combine(xf, sfc1_w, sfc1_b, sfc2_w, sfc2_b, ypair, top_w):
    n = xf.shape[0]
    nblk = n // _TB
    return pl.pallas_call(
        _combine_body,
        grid=(nblk,),
        in_specs=[
            pl.BlockSpec((_TB, _D), lambda b: (b, 0)),
            pl.BlockSpec((_INTER, _D), lambda b: (0, 0)),
            pl.BlockSpec((1, _INTER), lambda b: (0, 0)),
            pl.BlockSpec((_D, _INTER), lambda b: (0, 0)),
            pl.BlockSpec((1, _D), lambda b: (0, 0)),
            pl.BlockSpec((_TB, _DH), lambda b: (b, 0)),
            pl.BlockSpec((_TB, _DH), lambda b, _nb=nblk: (b + _nb, 0)),
            pl.BlockSpec((_TB, _TOPK), lambda b: (b, 0)),
        ],
        out_specs=pl.BlockSpec((_TB, _D), lambda b: (b, 0)),
        out_shape=jax.ShapeDtypeStruct((n, _D), jnp.float32),
    )(xf, sfc1_w, sfc1_b, sfc2_w, sfc2_b, ypair, ypair, top_w)


def kernel(x, gate_w, fc1_w, fc1_b, fc2_w, fc2_b, sfc1_w, sfc1_b, sfc2_w, sfc2_b):
    bb, hh, ww, dm = x.shape
    n = bb * hh * ww
    p = n * _TOPK
    xf = x.reshape(n, dm)

    row = jnp.arange(_GATE_TB, dtype=jnp.int32)
    tril = (row[:, None] > row[None, :]).astype(jnp.float32)
    top_idx, top_w, rank2, counts2, xbf = _gate(xf, gate_w, tril)

    counts = counts2.reshape(-1)
    blocks_e = (counts + _BLK - 1) // _BLK
    first_blk = jnp.cumsum(blocks_e) - blocks_e
    pad_offs = _BLK * first_blk
    block_expert = jnp.repeat(jnp.arange(_E, dtype=jnp.int32), blocks_e,
                              total_repeat_length=_NB)
    block_ord = jnp.arange(_NB, dtype=jnp.int32) - first_blk[block_expert]
    block_len = jnp.clip(counts[block_expert] - block_ord * _BLK, 0, _BLK)
    meta = jnp.concatenate([block_expert, block_len]).astype(jnp.int32)
    pp2 = _pp_compute(top_idx, rank2, pad_offs.reshape(1, _E))
    pp = pp2.T.reshape(-1)

    xg = _sc_dispatch(xbf, pp, _NB * _BLK)

    yg = _grouped_gemm(xg, fc1_w, fc1_b.reshape(_E, 1, _INTER),
                       fc2_w, fc2_b.reshape(_E, 1, _D), meta)

    ypair = _sc_undispatch(yg, pp)

    out = _combine(xf, sfc1_w, sfc1_b.reshape(1, _INTER),
                   sfc2_w, sfc2_b.reshape(1, _D), ypair, top_w)
    return out.reshape(bb, hh, ww, dm)


# speedup vs baseline: 1.1604x; 1.1223x over previous
import functools

import jax
import jax.numpy as jnp
from jax import lax
from jax.experimental import pallas as pl
from jax.experimental.pallas import tpu as pltpu
from jax.experimental.pallas import tpu_sc as plsc

_E = 64
_TOPK = 2
_D = 768
_INTER = 256
_BLK = 256
_NB = 128
_GATE_TB = 512
_TB = 512
_DH = _D // 2


def _pack_rows(xb):
    lo = lax.bitcast_convert_type(xb[:, :_DH], jnp.uint16).astype(jnp.uint32)
    hi = lax.bitcast_convert_type(xb[:, _DH:], jnp.uint16).astype(jnp.uint32)
    return lax.bitcast_convert_type(lo | (hi << 16), jnp.int32)


def _unpack_rows(u):
    u = lax.bitcast_convert_type(u, jnp.uint32)
    lo = lax.bitcast_convert_type((u & 0xFFFF).astype(jnp.uint16), jnp.bfloat16)
    hi = lax.bitcast_convert_type((u >> 16).astype(jnp.uint16), jnp.bfloat16)
    return jnp.concatenate([lo, hi], axis=1)


def _gate_body(x_ref, gw_ref, tril_ref, idx_ref, w_ref, rank_ref, cnt_ref,
               xbf_ref, carry_ref):
    b = pl.program_id(0)
    x = x_ref[...]
    xbf_ref[...] = _pack_rows(x.astype(jnp.bfloat16))
    g = gw_ref[...]
    logits = lax.dot_general(x, g, (((1,), (1,)), ((), ())),
                             preferred_element_type=jnp.float32)
    lanes = lax.broadcasted_iota(jnp.int32, logits.shape, 1)
    m1 = jnp.max(logits, axis=1, keepdims=True)
    i1 = jnp.min(jnp.where(logits == m1, lanes, _E), axis=1, keepdims=True)
    masked = jnp.where(lanes == i1, -jnp.inf, logits)
    m2 = jnp.max(masked, axis=1, keepdims=True)
    i2 = jnp.min(jnp.where(masked == m2, lanes, _E), axis=1, keepdims=True)
    w1 = 1.0 / (1.0 + jnp.exp(m2 - m1))
    w2 = 1.0 - w1
    idx_ref[...] = jnp.concatenate([i1, i2], axis=1)
    w_ref[...] = jnp.concatenate([w1, w2], axis=1)

    @pl.when(b == 0)
    def _():
        carry_ref[...] = jnp.zeros((1, _E), jnp.float32)

    carry = carry_ref[...]
    oh1 = (lanes == i1).astype(jnp.float32)
    oh2 = (lanes == i2).astype(jnp.float32)
    s = oh1 + oh2
    pex = lax.dot_general(tril_ref[...], s, (((1,), (0,)), ((), ())),
                          preferred_element_type=jnp.float32)
    r0 = jnp.sum((pex + carry) * oh1, axis=1, keepdims=True)
    r1 = jnp.sum((pex + carry) * oh2, axis=1, keepdims=True)
    rank_ref[...] = jnp.concatenate([r0, r1], axis=1).astype(jnp.int32)
    new_carry = carry + jnp.sum(s, axis=0, keepdims=True)
    carry_ref[...] = new_carry
    cnt_ref[...] = new_carry.astype(jnp.int32)


def _gate(xf, gate_w, tril):
    n = xf.shape[0]
    grid = (n // _GATE_TB,)
    return pl.pallas_call(
        _gate_body,
        grid=grid,
        in_specs=[
            pl.BlockSpec((_GATE_TB, _D), lambda b: (b, 0)),
            pl.BlockSpec((_E, _D), lambda b: (0, 0)),
            pl.BlockSpec((_GATE_TB, _GATE_TB), lambda b: (0, 0)),
        ],
        out_specs=[
            pl.BlockSpec((_GATE_TB, _TOPK), lambda b: (b, 0)),
            pl.BlockSpec((_GATE_TB, _TOPK), lambda b: (b, 0)),
            pl.BlockSpec((_GATE_TB, _TOPK), lambda b: (b, 0)),
            pl.BlockSpec((1, _E), lambda b: (0, 0)),
            pl.BlockSpec((_GATE_TB, _DH), lambda b: (b, 0)),
        ],
        out_shape=[
            jax.ShapeDtypeStruct((n, _TOPK), jnp.int32),
            jax.ShapeDtypeStruct((n, _TOPK), jnp.float32),
            jax.ShapeDtypeStruct((n, _TOPK), jnp.int32),
            jax.ShapeDtypeStruct((1, _E), jnp.int32),
            jax.ShapeDtypeStruct((n, _DH), jnp.int32),
        ],
        scratch_shapes=[pltpu.VMEM((1, _E), jnp.float32)],
    )(xf, gate_w, tril)


def _pp_body(idx_ref, rank_ref, po_ref, pp_ref):
    idx = idx_ref[...]
    lanes = lax.broadcasted_iota(jnp.int32, (idx.shape[0], _E), 1)
    po = po_ref[...].astype(jnp.float32)
    oh1 = (lanes == idx[:, 0:1]).astype(jnp.float32)
    oh2 = (lanes == idx[:, 1:2]).astype(jnp.float32)
    p0 = jnp.sum(oh1 * po, axis=1, keepdims=True)
    p1 = jnp.sum(oh2 * po, axis=1, keepdims=True)
    pp_ref[...] = (jnp.concatenate([p0, p1], axis=1).astype(jnp.int32)
                   + rank_ref[...])


def _pp_compute(top_idx, rank2, pad_offs):
    n = top_idx.shape[0]
    return pl.pallas_call(
        _pp_body,
        grid=(1,),
        in_specs=[
            pl.BlockSpec((n, _TOPK), lambda b: (0, 0)),
            pl.BlockSpec((n, _TOPK), lambda b: (0, 0)),
            pl.BlockSpec((1, _E), lambda b: (0, 0)),
        ],
        out_specs=pl.BlockSpec((n, _TOPK), lambda b: (0, 0)),
        out_shape=jax.ShapeDtypeStruct((n, _TOPK), jnp.int32),
    )(top_idx, rank2, pad_offs)


_CH = 128


def _sc_dispatch(table, pp, out_rows):
    b, = pp.shape
    n, d = table.shape
    nw = 32
    per_w = b // nw
    n_ch = per_w // _CH
    mesh = plsc.VectorSubcoreMesh(core_axis_name="c", subcore_axis_name="s")

    @functools.partial(
        pl.kernel,
        mesh=mesh,
        out_type=jax.ShapeDtypeStruct((out_rows, d), table.dtype),
        scratch_types=[
            pltpu.VMEM((_CH,), jnp.int32),
            pltpu.VMEM((_CH, d), table.dtype),
            pltpu.SemaphoreType.DMA,
        ],
    )
    def k(table_hbm, pp_hbm, out_hbm, pp_v, rows_v, sem):
        wid = lax.axis_index("s") * 2 + lax.axis_index("c")

        def body(c, carry):
            base = wid * per_w + c * _CH
            tok_base = base - (base // n) * n
            pltpu.sync_copy(pp_hbm.at[pl.ds(base, _CH)], pp_v)
            pltpu.sync_copy(table_hbm.at[pl.ds(tok_base, _CH)], rows_v)
            pltpu.async_copy(rows_v, out_hbm.at[pp_v], sem).wait()
            return carry

        lax.fori_loop(0, n_ch, body, 0)

    return k(table, pp)


def _sc_undispatch(table, pp):
    b, = pp.shape
    d = table.shape[1]
    nw = 32
    per_w = b // nw
    n_ch = per_w // _CH
    mesh = plsc.VectorSubcoreMesh(core_axis_name="c", subcore_axis_name="s")

    @functools.partial(
        pl.kernel,
        mesh=mesh,
        out_type=jax.ShapeDtypeStruct((b, d), table.dtype),
        scratch_types=[
            pltpu.VMEM((_CH,), jnp.int32),
            pltpu.VMEM((_CH, d), table.dtype),
            pltpu.SemaphoreType.DMA,
        ],
    )
    def k(table_hbm, pp_hbm, out_hbm, pp_v, rows_v, sem):
        wid = lax.axis_index("s") * 2 + lax.axis_index("c")

        def body(c, carry):
            base = wid * per_w + c * _CH
            pltpu.sync_copy(pp_hbm.at[pl.ds(base, _CH)], pp_v)
            pltpu.async_copy(table_hbm.at[pp_v], rows_v, sem).wait()
            pltpu.sync_copy(rows_v, out_hbm.at[pl.ds(base, _CH)])
            return carry

        lax.fori_loop(0, n_ch, body, 0)

    return k(table, pp)


def _expert_mlp(xi, blen, w1_ref, b1_ref, w2_ref, b2_ref):
    x = _unpack_rows(xi)
    rows = lax.broadcasted_iota(jnp.int32, (_BLK, 1), 0)
    xb = jnp.where(rows < blen, x, jnp.bfloat16(0))
    w1 = w1_ref[0].astype(jnp.bfloat16)
    h = lax.dot_general(xb, w1, (((1,), (1,)), ((), ())),
                        preferred_element_type=jnp.float32)
    h = h + b1_ref[0]
    h = 0.5 * h * (1.0 + lax.erf(h * 0.7071067811865476))
    hb = h.astype(jnp.bfloat16)
    w2 = w2_ref[0].astype(jnp.bfloat16)
    y = lax.dot_general(hb, w2, (((1,), (1,)), ((), ())),
                        preferred_element_type=jnp.float32)
    return _pack_rows((y + b2_ref[0]).astype(jnp.bfloat16))


_NWAY = 4


def _ggemm_body(meta_ref, x_ref, *refs):
    b = pl.program_id(0)
    nq = _NB // _NWAY
    ws, out_ref = refs[:-1], refs[-1]
    xx = x_ref[...]
    for q in range(_NWAY):
        w1_ref, b1_ref, w2_ref, b2_ref = ws[4 * q:4 * q + 4]
        out_ref[q, 0] = _expert_mlp(xx[q, 0], meta_ref[_NB + q * nq + b],
                                    w1_ref, b1_ref, w2_ref, b2_ref)


def _grouped_gemm(xg, fc1_w, fc1_b, fc2_w, fc2_b, meta):
    nq = _NB // _NWAY
    wspecs = []
    for q in range(_NWAY):
        wspecs += [
            pl.BlockSpec((1, _INTER, _D), lambda b, m, _o=q * nq: (m[_o + b], 0, 0)),
            pl.BlockSpec((1, 1, _INTER), lambda b, m, _o=q * nq: (m[_o + b], 0, 0)),
            pl.BlockSpec((1, _D, _INTER), lambda b, m, _o=q * nq: (m[_o + b], 0, 0)),
            pl.BlockSpec((1, 1, _D), lambda b, m, _o=q * nq: (m[_o + b], 0, 0)),
        ]
    grid_spec = pltpu.PrefetchScalarGridSpec(
        num_scalar_prefetch=1,
        grid=(nq,),
        in_specs=[pl.BlockSpec((_NWAY, 1, _BLK, _DH),
                               lambda b, m: (0, b, 0, 0))] + wspecs,
        out_specs=pl.BlockSpec((_NWAY, 1, _BLK, _DH), lambda b, m: (0, b, 0, 0)),
    )
    xg4 = xg.reshape(_NWAY, nq, _BLK, _DH)
    out = pl.pallas_call(
        _ggemm_body,
        grid_spec=grid_spec,
        out_shape=jax.ShapeDtypeStruct((_NWAY, nq, _BLK, _DH), jnp.int32),
    )(meta, xg4, *([fc1_w, fc1_b, fc2_w, fc2_b] * _NWAY))
    return out.reshape(_NB * _BLK, _DH)


def _combine_body(x_ref, w1_ref, b1_ref, w2_ref, b2_ref, y0_ref, y1_ref,
                  tw_ref, out_ref):
    x = x_ref[...]
    h = lax.dot_general(x, w1_ref[...], (((1,), (1,)), ((), ())),
                        preferred_element_type=jnp.float32)
    h = h + b1_ref[...]
    h = 0.5 * h * (1.0 + lax.erf(h * 0.7071067811865476))
    s = lax.dot_general(h, w2_ref[...], (((1,), (1,)), ((), ())),
                        preferred_element_type=jnp.float32)
    s = s + b2_ref[...]
    tw = tw_ref[...]
    y0 = _unpack_rows(y0_ref[...]).astype(jnp.float32)
    y1 = _unpack_rows(y1_ref[...]).astype(jnp.float32)
    out_ref[...] = s + tw[:, 0:1] * y0 + tw[:, 1:2] * y1


def _combine(xf, sfc1_w, sfc1_b, sfc2_w, sfc2_b, ypair, top_w):
    n = xf.shape[0]
    nblk = n // _TB
    return pl.pallas_call(
        _combine_body,
        grid=(nblk,),
        in_specs=[
            pl.BlockSpec((_TB, _D), lambda b: (b, 0)),
            pl.BlockSpec((_INTER, _D), lambda b: (0, 0)),
            pl.BlockSpec((1, _INTER), lambda b: (0, 0)),
            pl.BlockSpec((_D, _INTER), lambda b: (0, 0)),
            pl.BlockSpec((1, _D), lambda b: (0, 0)),
            pl.BlockSpec((_TB, _DH), lambda b: (b, 0)),
            pl.BlockSpec((_TB, _DH), lambda b, _nb=nblk: (b + _nb, 0)),
            pl.BlockSpec((_TB, _TOPK), lambda b: (b, 0)),
        ],
        out_specs=pl.BlockSpec((_TB, _D), lambda b: (b, 0)),
        out_shape=jax.ShapeDtypeStruct((n, _D), jnp.float32),
    )(xf, sfc1_w, sfc1_b, sfc2_w, sfc2_b, ypair, ypair, top_w)


def kernel(x, gate_w, fc1_w, fc1_b, fc2_w, fc2_b, sfc1_w, sfc1_b, sfc2_w, sfc2_b):
    bb, hh, ww, dm = x.shape
    n = bb * hh * ww
    p = n * _TOPK
    xf = x.reshape(n, dm)

    row = jnp.arange(_GATE_TB, dtype=jnp.int32)
    tril = (row[:, None] > row[None, :]).astype(jnp.float32)
    top_idx, top_w, rank2, counts2, xbf = _gate(xf, gate_w, tril)

    counts = counts2.reshape(-1)
    blocks_e = (counts + _BLK - 1) // _BLK
    first_blk = jnp.cumsum(blocks_e) - blocks_e
    pad_offs = _BLK * first_blk
    block_expert = jnp.repeat(jnp.arange(_E, dtype=jnp.int32), blocks_e,
                              total_repeat_length=_NB)
    block_ord = jnp.arange(_NB, dtype=jnp.int32) - first_blk[block_expert]
    block_len = jnp.clip(counts[block_expert] - block_ord * _BLK, 0, _BLK)
    meta = jnp.concatenate([block_expert, block_len]).astype(jnp.int32)
    pp2 = _pp_compute(top_idx, rank2, pad_offs.reshape(1, _E))
    pp = pp2.T.reshape(-1)

    xg = _sc_dispatch(xbf, pp, _NB * _BLK)

    yg = _grouped_gemm(xg, fc1_w, fc1_b.reshape(_E, 1, _INTER),
                       fc2_w, fc2_b.reshape(_E, 1, _D), meta)

    ypair = _sc_undispatch(yg, pp)

    out = _combine(xf, sfc1_w, sfc1_b.reshape(1, _INTER),
                   sfc2_w, sfc2_b.reshape(1, _D), ypair, top_w)
    return out.reshape(bb, hh, ww, dm)


# speedup vs baseline: 1.1908x; 1.0262x over previous
import functools

import jax
import jax.numpy as jnp
from jax import lax
from jax.experimental import pallas as pl
from jax.experimental.pallas import tpu as pltpu
from jax.experimental.pallas import tpu_sc as plsc

_E = 64
_TOPK = 2
_D = 768
_INTER = 256
_BLK = 256
_NB = 128
_GATE_TB = 512
_TB = 512
_DH = _D // 2


def _pack_rows(xb):
    lo = lax.bitcast_convert_type(xb[:, :_DH], jnp.uint16).astype(jnp.uint32)
    hi = lax.bitcast_convert_type(xb[:, _DH:], jnp.uint16).astype(jnp.uint32)
    return lax.bitcast_convert_type(lo | (hi << 16), jnp.int32)


def _unpack_rows(u):
    u = lax.bitcast_convert_type(u, jnp.uint32)
    lo = lax.bitcast_convert_type((u & 0xFFFF).astype(jnp.uint16), jnp.bfloat16)
    hi = lax.bitcast_convert_type((u >> 16).astype(jnp.uint16), jnp.bfloat16)
    return jnp.concatenate([lo, hi], axis=1)


def _gate_body(x_ref, gw_ref, tril_ref, idx_ref, w_ref, rank_ref, cnt_ref,
               xbf_ref, carry_ref):
    b = pl.program_id(0)
    x = x_ref[...]
    xbf_ref[...] = _pack_rows(x.astype(jnp.bfloat16))
    g = gw_ref[...]
    logits = lax.dot_general(x, g, (((1,), (1,)), ((), ())),
                             preferred_element_type=jnp.float32)
    lanes = lax.broadcasted_iota(jnp.int32, logits.shape, 1)
    m1 = jnp.max(logits, axis=1, keepdims=True)
    i1 = jnp.min(jnp.where(logits == m1, lanes, _E), axis=1, keepdims=True)
    masked = jnp.where(lanes == i1, -jnp.inf, logits)
    m2 = jnp.max(masked, axis=1, keepdims=True)
    i2 = jnp.min(jnp.where(masked == m2, lanes, _E), axis=1, keepdims=True)
    w1 = 1.0 / (1.0 + jnp.exp(m2 - m1))
    w2 = 1.0 - w1
    idx_ref[...] = jnp.concatenate([i1, i2], axis=1)
    w_ref[...] = jnp.concatenate([w1, w2], axis=1)

    @pl.when(b == 0)
    def _():
        carry_ref[...] = jnp.zeros((1, _E), jnp.float32)

    carry = carry_ref[...]
    oh1 = (lanes == i1).astype(jnp.float32)
    oh2 = (lanes == i2).astype(jnp.float32)
    s = oh1 + oh2
    pex = lax.dot_general(tril_ref[...], s, (((1,), (0,)), ((), ())),
                          preferred_element_type=jnp.float32)
    r0 = jnp.sum((pex + carry) * oh1, axis=1, keepdims=True)
    r1 = jnp.sum((pex + carry) * oh2, axis=1, keepdims=True)
    rank_ref[...] = jnp.concatenate([r0, r1], axis=1).astype(jnp.int32)
    new_carry = carry + jnp.sum(s, axis=0, keepdims=True)
    carry_ref[...] = new_carry
    cnt_ref[...] = new_carry.astype(jnp.int32)


def _gate(xf, gate_w, tril):
    n = xf.shape[0]
    grid = (n // _GATE_TB,)
    return pl.pallas_call(
        _gate_body,
        grid=grid,
        in_specs=[
            pl.BlockSpec((_GATE_TB, _D), lambda b: (b, 0)),
            pl.BlockSpec((_E, _D), lambda b: (0, 0)),
            pl.BlockSpec((_GATE_TB, _GATE_TB), lambda b: (0, 0)),
        ],
        out_specs=[
            pl.BlockSpec((_GATE_TB, _TOPK), lambda b: (b, 0)),
            pl.BlockSpec((_GATE_TB, _TOPK), lambda b: (b, 0)),
            pl.BlockSpec((_GATE_TB, _TOPK), lambda b: (b, 0)),
            pl.BlockSpec((1, _E), lambda b: (0, 0)),
            pl.BlockSpec((_GATE_TB, _DH), lambda b: (b, 0)),
        ],
        out_shape=[
            jax.ShapeDtypeStruct((n, _TOPK), jnp.int32),
            jax.ShapeDtypeStruct((n, _TOPK), jnp.float32),
            jax.ShapeDtypeStruct((n, _TOPK), jnp.int32),
            jax.ShapeDtypeStruct((1, _E), jnp.int32),
            jax.ShapeDtypeStruct((n, _DH), jnp.int32),
        ],
        scratch_shapes=[pltpu.VMEM((1, _E), jnp.float32)],
    )(xf, gate_w, tril)


def _pp_body(idx_ref, rank_ref, po_ref, pp_ref):
    idx = idx_ref[...]
    lanes = lax.broadcasted_iota(jnp.int32, (idx.shape[0], _E), 1)
    po = po_ref[...].astype(jnp.float32)
    oh1 = (lanes == idx[:, 0:1]).astype(jnp.float32)
    oh2 = (lanes == idx[:, 1:2]).astype(jnp.float32)
    p0 = jnp.sum(oh1 * po, axis=1, keepdims=True)
    p1 = jnp.sum(oh2 * po, axis=1, keepdims=True)
    pp_ref[...] = (jnp.concatenate([p0, p1], axis=1).astype(jnp.int32)
                   + rank_ref[...])


def _pp_compute(top_idx, rank2, pad_offs):
    n = top_idx.shape[0]
    return pl.pallas_call(
        _pp_body,
        grid=(1,),
        in_specs=[
            pl.BlockSpec((n, _TOPK), lambda b: (0, 0)),
            pl.BlockSpec((n, _TOPK), lambda b: (0, 0)),
            pl.BlockSpec((1, _E), lambda b: (0, 0)),
        ],
        out_specs=pl.BlockSpec((n, _TOPK), lambda b: (0, 0)),
        out_shape=jax.ShapeDtypeStruct((n, _TOPK), jnp.int32),
    )(top_idx, rank2, pad_offs)


_CH = 128


def _sc_dispatch(table, pp, out_rows):
    b, = pp.shape
    n, d = table.shape
    nw = 32
    per_w = b // nw
    n_ch = per_w // _CH
    mesh = plsc.VectorSubcoreMesh(core_axis_name="c", subcore_axis_name="s")

    @functools.partial(
        pl.kernel,
        mesh=mesh,
        out_type=jax.ShapeDtypeStruct((out_rows, d), table.dtype),
        scratch_types=[
            pltpu.VMEM((_CH,), jnp.int32),
            pltpu.VMEM((_CH, d), table.dtype),
            pltpu.SemaphoreType.DMA,
        ],
    )
    def k(table_hbm, pp_hbm, out_hbm, pp_v, rows_v, sem):
        wid = lax.axis_index("s") * 2 + lax.axis_index("c")

        def body(c, carry):
            base = wid * per_w + c * _CH
            tok_base = base - (base // n) * n
            pltpu.sync_copy(pp_hbm.at[pl.ds(base, _CH)], pp_v)
            pltpu.sync_copy(table_hbm.at[pl.ds(tok_base, _CH)], rows_v)
            pltpu.async_copy(rows_v, out_hbm.at[pp_v], sem).wait()
            return carry

        lax.fori_loop(0, n_ch, body, 0)

    return k(table, pp)


def _sc_undispatch(table, pp):
    b, = pp.shape
    d = table.shape[1]
    nw = 32
    per_w = b // nw
    n_ch = per_w // _CH
    mesh = plsc.VectorSubcoreMesh(core_axis_name="c", subcore_axis_name="s")

    @functools.partial(
        pl.kernel,
        mesh=mesh,
        out_type=jax.ShapeDtypeStruct((b, d), table.dtype),
        scratch_types=[
            pltpu.VMEM((_CH,), jnp.int32),
            pltpu.VMEM((_CH, d), table.dtype),
            pltpu.SemaphoreType.DMA,
        ],
    )
    def k(table_hbm, pp_hbm, out_hbm, pp_v, rows_v, sem):
        wid = lax.axis_index("s") * 2 + lax.axis_index("c")

        def body(c, carry):
            base = wid * per_w + c * _CH
            pltpu.sync_copy(pp_hbm.at[pl.ds(base, _CH)], pp_v)
            pltpu.async_copy(table_hbm.at[pp_v], rows_v, sem).wait()
            pltpu.sync_copy(rows_v, out_hbm.at[pl.ds(base, _CH)])
            return carry

        lax.fori_loop(0, n_ch, body, 0)

    return k(table, pp)


def _expert_mlp(xi, blen, w1_ref, b1_ref, w2_ref, b2_ref):
    x = _unpack_rows(xi)
    rows = lax.broadcasted_iota(jnp.int32, (_BLK, 1), 0)
    xb = jnp.where(rows < blen, x, jnp.bfloat16(0))
    w1 = w1_ref[0].astype(jnp.bfloat16)
    h = lax.dot_general(xb, w1, (((1,), (1,)), ((), ())),
                        preferred_element_type=jnp.float32)
    h = h + b1_ref[0]
    h = 0.5 * h * (1.0 + lax.erf(h * 0.7071067811865476))
    hb = h.astype(jnp.bfloat16)
    w2 = w2_ref[0].astype(jnp.bfloat16)
    y = lax.dot_general(hb, w2, (((1,), (1,)), ((), ())),
                        preferred_element_type=jnp.float32)
    return _pack_rows((y + b2_ref[0]).astype(jnp.bfloat16))


_NWAY = 8


def _ggemm_body(meta_ref, x_ref, *refs):
    b = pl.program_id(0)
    nq = _NB // _NWAY
    ws, out_ref = refs[:-1], refs[-1]
    xx = x_ref[...]
    for q in range(_NWAY):
        w1_ref, b1_ref, w2_ref, b2_ref = ws[4 * q:4 * q + 4]
        out_ref[q, 0] = _expert_mlp(xx[q, 0], meta_ref[_NB + q * nq + b],
                                    w1_ref, b1_ref, w2_ref, b2_ref)


def _grouped_gemm(xg, fc1_w, fc1_b, fc2_w, fc2_b, meta):
    nq = _NB // _NWAY
    wspecs = []
    for q in range(_NWAY):
        wspecs += [
            pl.BlockSpec((1, _INTER, _D), lambda b, m, _o=q * nq: (m[_o + b], 0, 0)),
            pl.BlockSpec((1, 1, _INTER), lambda b, m, _o=q * nq: (m[_o + b], 0, 0)),
            pl.BlockSpec((1, _D, _INTER), lambda b, m, _o=q * nq: (m[_o + b], 0, 0)),
            pl.BlockSpec((1, 1, _D), lambda b, m, _o=q * nq: (m[_o + b], 0, 0)),
        ]
    grid_spec = pltpu.PrefetchScalarGridSpec(
        num_scalar_prefetch=1,
        grid=(nq,),
        in_specs=[pl.BlockSpec((_NWAY, 1, _BLK, _DH),
                               lambda b, m: (0, b, 0, 0))] + wspecs,
        out_specs=pl.BlockSpec((_NWAY, 1, _BLK, _DH), lambda b, m: (0, b, 0, 0)),
    )
    xg4 = xg.reshape(_NWAY, nq, _BLK, _DH)
    out = pl.pallas_call(
        _ggemm_body,
        grid_spec=grid_spec,
        out_shape=jax.ShapeDtypeStruct((_NWAY, nq, _BLK, _DH), jnp.int32),
    )(meta, xg4, *([fc1_w, fc1_b, fc2_w, fc2_b] * _NWAY))
    return out.reshape(_NB * _BLK, _DH)


def _combine_body(x_ref, w1_ref, b1_ref, w2_ref, b2_ref, y0_ref, y1_ref,
                  tw_ref, out_ref):
    x = x_ref[...]
    h = lax.dot_general(x, w1_ref[...], (((1,), (1,)), ((), ())),
                        preferred_element_type=jnp.float32)
    h = h + b1_ref[...]
    h = 0.5 * h * (1.0 + lax.erf(h * 0.7071067811865476))
    s = lax.dot_general(h, w2_ref[...], (((1,), (1,)), ((), ())),
                        preferred_element_type=jnp.float32)
    s = s + b2_ref[...]
    tw = tw_ref[...]
    y0 = _unpack_rows(y0_ref[...]).astype(jnp.float32)
    y1 = _unpack_rows(y1_ref[...]).astype(jnp.float32)
    out_ref[...] = s + tw[:, 0:1] * y0 + tw[:, 1:2] * y1


def _combine(xf, sfc1_w, sfc1_b, sfc2_w, sfc2_b, ypair, top_w):
    n = xf.shape[0]
    nblk = n // _TB
    return pl.pallas_call(
        _combine_body,
        grid=(nblk,),
        in_specs=[
            pl.BlockSpec((_TB, _D), lambda b: (b, 0)),
            pl.BlockSpec((_INTER, _D), lambda b: (0, 0)),
            pl.BlockSpec((1, _INTER), lambda b: (0, 0)),
            pl.BlockSpec((_D, _INTER), lambda b: (0, 0)),
            pl.BlockSpec((1, _D), lambda b: (0, 0)),
            pl.BlockSpec((_TB, _DH), lambda b: (b, 0)),
            pl.BlockSpec((_TB, _DH), lambda b, _nb=nblk: (b + _nb, 0)),
            pl.BlockSpec((_TB, _TOPK), lambda b: (b, 0)),
        ],
        out_specs=pl.BlockSpec((_TB, _D), lambda b: (b, 0)),
        out_shape=jax.ShapeDtypeStruct((n, _D), jnp.float32),
    )(xf, sfc1_w, sfc1_b, sfc2_w, sfc2_b, ypair, ypair, top_w)


def kernel(x, gate_w, fc1_w, fc1_b, fc2_w, fc2_b, sfc1_w, sfc1_b, sfc2_w, sfc2_b):
    bb, hh, ww, dm = x.shape
    n = bb * hh * ww
    p = n * _TOPK
    xf = x.reshape(n, dm)

    row = jnp.arange(_GATE_TB, dtype=jnp.int32)
    tril = (row[:, None] > row[None, :]).astype(jnp.float32)
    top_idx, top_w, rank2, counts2, xbf = _gate(xf, gate_w, tril)

    counts = counts2.reshape(-1)
    blocks_e = (counts + _BLK - 1) // _BLK
    first_blk = jnp.cumsum(blocks_e) - blocks_e
    pad_offs = _BLK * first_blk
    block_expert = jnp.repeat(jnp.arange(_E, dtype=jnp.int32), blocks_e,
                              total_repeat_length=_NB)
    block_ord = jnp.arange(_NB, dtype=jnp.int32) - first_blk[block_expert]
    block_len = jnp.clip(counts[block_expert] - block_ord * _BLK, 0, _BLK)
    meta = jnp.concatenate([block_expert, block_len]).astype(jnp.int32)
    pp2 = _pp_compute(top_idx, rank2, pad_offs.reshape(1, _E))
    pp = pp2.T.reshape(-1)

    xg = _sc_dispatch(xbf, pp, _NB * _BLK)

    yg = _grouped_gemm(xg, fc1_w, fc1_b.reshape(_E, 1, _INTER),
                       fc2_w, fc2_b.reshape(_E, 1, _D), meta)

    ypair = _sc_undispatch(yg, pp)

    out = _combine(xf, sfc1_w, sfc1_b.reshape(1, _INTER),
                   sfc2_w, sfc2_b.reshape(1, _D), ypair, top_w)
    return out.reshape(bb, hh, ww, dm)


# speedup vs baseline: 1.2023x; 1.0096x over previous
import functools

import jax
import jax.numpy as jnp
from jax import lax
from jax.experimental import pallas as pl
from jax.experimental.pallas import tpu as pltpu
from jax.experimental.pallas import tpu_sc as plsc

_E = 64
_TOPK = 2
_D = 768
_INTER = 256
_BLK = 256
_NB = 128
_GATE_TB = 512
_TB = 512
_DH = _D // 2


def _pack_rows(xb):
    lo = lax.bitcast_convert_type(xb[:, :_DH], jnp.uint16).astype(jnp.uint32)
    hi = lax.bitcast_convert_type(xb[:, _DH:], jnp.uint16).astype(jnp.uint32)
    return lax.bitcast_convert_type(lo | (hi << 16), jnp.int32)


def _unpack_rows(u):
    u = lax.bitcast_convert_type(u, jnp.uint32)
    lo = lax.bitcast_convert_type((u & 0xFFFF).astype(jnp.uint16), jnp.bfloat16)
    hi = lax.bitcast_convert_type((u >> 16).astype(jnp.uint16), jnp.bfloat16)
    return jnp.concatenate([lo, hi], axis=1)


def _gate_body(x_ref, gw_ref, tril_ref, idx_ref, w_ref, rank_ref, cnt_ref,
               xbf_ref, carry_ref):
    b = pl.program_id(0)
    x = x_ref[...]
    xbf_ref[...] = _pack_rows(x.astype(jnp.bfloat16))
    g = gw_ref[...]
    logits = lax.dot_general(x, g, (((1,), (1,)), ((), ())),
                             preferred_element_type=jnp.float32)
    lanes = lax.broadcasted_iota(jnp.int32, logits.shape, 1)
    m1 = jnp.max(logits, axis=1, keepdims=True)
    i1 = jnp.min(jnp.where(logits == m1, lanes, _E), axis=1, keepdims=True)
    masked = jnp.where(lanes == i1, -jnp.inf, logits)
    m2 = jnp.max(masked, axis=1, keepdims=True)
    i2 = jnp.min(jnp.where(masked == m2, lanes, _E), axis=1, keepdims=True)
    w1 = 1.0 / (1.0 + jnp.exp(m2 - m1))
    w2 = 1.0 - w1
    idx_ref[...] = jnp.concatenate([i1, i2], axis=1)
    w_ref[...] = jnp.concatenate([w1, w2], axis=1)

    @pl.when(b == 0)
    def _():
        carry_ref[...] = jnp.zeros((1, _E), jnp.float32)

    carry = carry_ref[...]
    oh1 = (lanes == i1).astype(jnp.float32)
    oh2 = (lanes == i2).astype(jnp.float32)
    s = oh1 + oh2
    pex = lax.dot_general(tril_ref[...], s, (((1,), (0,)), ((), ())),
                          preferred_element_type=jnp.float32)
    r0 = jnp.sum((pex + carry) * oh1, axis=1, keepdims=True)
    r1 = jnp.sum((pex + carry) * oh2, axis=1, keepdims=True)
    rank_ref[...] = jnp.concatenate([r0, r1], axis=1).astype(jnp.int32)
    new_carry = carry + jnp.sum(s, axis=0, keepdims=True)
    carry_ref[...] = new_carry
    cnt_ref[...] = new_carry.astype(jnp.int32)


def _gate(xf, gate_w, tril):
    n = xf.shape[0]
    grid = (n // _GATE_TB,)
    return pl.pallas_call(
        _gate_body,
        grid=grid,
        in_specs=[
            pl.BlockSpec((_GATE_TB, _D), lambda b: (b, 0)),
            pl.BlockSpec((_E, _D), lambda b: (0, 0)),
            pl.BlockSpec((_GATE_TB, _GATE_TB), lambda b: (0, 0)),
        ],
        out_specs=[
            pl.BlockSpec((_GATE_TB, _TOPK), lambda b: (b, 0)),
            pl.BlockSpec((_GATE_TB, _TOPK), lambda b: (b, 0)),
            pl.BlockSpec((_GATE_TB, _TOPK), lambda b: (b, 0)),
            pl.BlockSpec((1, _E), lambda b: (0, 0)),
            pl.BlockSpec((_GATE_TB, _DH), lambda b: (b, 0)),
        ],
        out_shape=[
            jax.ShapeDtypeStruct((n, _TOPK), jnp.int32),
            jax.ShapeDtypeStruct((n, _TOPK), jnp.float32),
            jax.ShapeDtypeStruct((n, _TOPK), jnp.int32),
            jax.ShapeDtypeStruct((1, _E), jnp.int32),
            jax.ShapeDtypeStruct((n, _DH), jnp.int32),
        ],
        scratch_shapes=[pltpu.VMEM((1, _E), jnp.float32)],
    )(xf, gate_w, tril)


def _pp_body(idx_ref, rank_ref, po_ref, pp_ref):
    idx = idx_ref[...]
    lanes = lax.broadcasted_iota(jnp.int32, (idx.shape[0], _E), 1)
    po = po_ref[...].astype(jnp.float32)
    oh1 = (lanes == idx[:, 0:1]).astype(jnp.float32)
    oh2 = (lanes == idx[:, 1:2]).astype(jnp.float32)
    p0 = jnp.sum(oh1 * po, axis=1, keepdims=True)
    p1 = jnp.sum(oh2 * po, axis=1, keepdims=True)
    pp_ref[...] = (jnp.concatenate([p0, p1], axis=1).astype(jnp.int32)
                   + rank_ref[...])


def _pp_compute(top_idx, rank2, pad_offs):
    n = top_idx.shape[0]
    return pl.pallas_call(
        _pp_body,
        grid=(1,),
        in_specs=[
            pl.BlockSpec((n, _TOPK), lambda b: (0, 0)),
            pl.BlockSpec((n, _TOPK), lambda b: (0, 0)),
            pl.BlockSpec((1, _E), lambda b: (0, 0)),
        ],
        out_specs=pl.BlockSpec((n, _TOPK), lambda b: (0, 0)),
        out_shape=jax.ShapeDtypeStruct((n, _TOPK), jnp.int32),
    )(top_idx, rank2, pad_offs)


_CH = 128


def _sc_dispatch(table, pp, out_rows):
    b, = pp.shape
    n, d = table.shape
    nw = 32
    per_w = b // nw
    n_ch = per_w // _CH
    mesh = plsc.VectorSubcoreMesh(core_axis_name="c", subcore_axis_name="s")

    @functools.partial(
        pl.kernel,
        mesh=mesh,
        out_type=jax.ShapeDtypeStruct((out_rows, d), table.dtype),
        scratch_types=[
            pltpu.VMEM((_CH,), jnp.int32),
            pltpu.VMEM((_CH, d), table.dtype),
            pltpu.SemaphoreType.DMA,
        ],
    )
    def k(table_hbm, pp_hbm, out_hbm, pp_v, rows_v, sem):
        wid = lax.axis_index("s") * 2 + lax.axis_index("c")

        def body(c, carry):
            base = wid * per_w + c * _CH
            tok_base = base - (base // n) * n
            pltpu.sync_copy(pp_hbm.at[pl.ds(base, _CH)], pp_v)
            pltpu.sync_copy(table_hbm.at[pl.ds(tok_base, _CH)], rows_v)
            pltpu.async_copy(rows_v, out_hbm.at[pp_v], sem).wait()
            return carry

        lax.fori_loop(0, n_ch, body, 0)

    return k(table, pp)


def _sc_undispatch(table, pp):
    b, = pp.shape
    d = table.shape[1]
    nw = 32
    per_w = b // nw
    n_ch = per_w // _CH
    mesh = plsc.VectorSubcoreMesh(core_axis_name="c", subcore_axis_name="s")

    @functools.partial(
        pl.kernel,
        mesh=mesh,
        out_type=jax.ShapeDtypeStruct((b, d), table.dtype),
        scratch_types=[
            pltpu.VMEM((_CH,), jnp.int32),
            pltpu.VMEM((_CH, d), table.dtype),
            pltpu.SemaphoreType.DMA,
        ],
    )
    def k(table_hbm, pp_hbm, out_hbm, pp_v, rows_v, sem):
        wid = lax.axis_index("s") * 2 + lax.axis_index("c")

        def body(c, carry):
            base = wid * per_w + c * _CH
            pltpu.sync_copy(pp_hbm.at[pl.ds(base, _CH)], pp_v)
            pltpu.async_copy(table_hbm.at[pp_v], rows_v, sem).wait()
            pltpu.sync_copy(rows_v, out_hbm.at[pl.ds(base, _CH)])
            return carry

        lax.fori_loop(0, n_ch, body, 0)

    return k(table, pp)


def _expert_mlp(xi, blen, w1_ref, b1_ref, w2_ref, b2_ref):
    x = _unpack_rows(xi)
    rows = lax.broadcasted_iota(jnp.int32, (_BLK, 1), 0)
    xb = jnp.where(rows < blen, x, jnp.bfloat16(0))
    w1 = w1_ref[0].astype(jnp.bfloat16)
    h = lax.dot_general(xb, w1, (((1,), (1,)), ((), ())),
                        preferred_element_type=jnp.float32)
    h = h + b1_ref[0]
    h = 0.5 * h * (1.0 + lax.erf(h * 0.7071067811865476))
    hb = h.astype(jnp.bfloat16)
    w2 = w2_ref[0].astype(jnp.bfloat16)
    y = lax.dot_general(hb, w2, (((1,), (1,)), ((), ())),
                        preferred_element_type=jnp.float32)
    return _pack_rows((y + b2_ref[0]).astype(jnp.bfloat16))


_NWAY = 8


def _ggemm_body(meta_ref, x_ref, *refs):
    b = pl.program_id(0)
    nq = _NB // _NWAY
    ws, out_ref = refs[:-1], refs[-1]
    xx = x_ref[...]
    for q in range(_NWAY):
        w1_ref, b1_ref, w2_ref, b2_ref = ws[4 * q:4 * q + 4]
        out_ref[q, 0] = _expert_mlp(xx[q, 0], meta_ref[_NB + q * nq + b],
                                    w1_ref, b1_ref, w2_ref, b2_ref)


def _grouped_gemm(xg, fc1_w, fc1_b, fc2_w, fc2_b, meta):
    nq = _NB // _NWAY
    wspecs = []
    for q in range(_NWAY):
        wspecs += [
            pl.BlockSpec((1, _INTER, _D), lambda b, m, _o=q * nq: (m[_o + b], 0, 0)),
            pl.BlockSpec((1, 1, _INTER), lambda b, m, _o=q * nq: (m[_o + b], 0, 0)),
            pl.BlockSpec((1, _D, _INTER), lambda b, m, _o=q * nq: (m[_o + b], 0, 0)),
            pl.BlockSpec((1, 1, _D), lambda b, m, _o=q * nq: (m[_o + b], 0, 0)),
        ]
    grid_spec = pltpu.PrefetchScalarGridSpec(
        num_scalar_prefetch=1,
        grid=(nq,),
        in_specs=[pl.BlockSpec((_NWAY, 1, _BLK, _DH),
                               lambda b, m: (0, b, 0, 0))] + wspecs,
        out_specs=pl.BlockSpec((_NWAY, 1, _BLK, _DH), lambda b, m: (0, b, 0, 0)),
    )
    xg4 = xg.reshape(_NWAY, nq, _BLK, _DH)
    out = pl.pallas_call(
        _ggemm_body,
        grid_spec=grid_spec,
        out_shape=jax.ShapeDtypeStruct((_NWAY, nq, _BLK, _DH), jnp.int32),
    )(meta, xg4, *([fc1_w, fc1_b, fc2_w, fc2_b] * _NWAY))
    return out.reshape(_NB * _BLK, _DH)


def _combine_body(x_ref, w1_ref, b1_ref, w2_ref, b2_ref, y0_ref, y1_ref,
                  tw_ref, out_ref):
    x = _unpack_rows(x_ref[...])
    h = lax.dot_general(x, w1_ref[...].astype(jnp.bfloat16),
                        (((1,), (1,)), ((), ())),
                        preferred_element_type=jnp.float32)
    h = h + b1_ref[...]
    h = 0.5 * h * (1.0 + lax.erf(h * 0.7071067811865476))
    s = lax.dot_general(h, w2_ref[...], (((1,), (1,)), ((), ())),
                        preferred_element_type=jnp.float32)
    s = s + b2_ref[...]
    tw = tw_ref[...]
    y0 = _unpack_rows(y0_ref[...]).astype(jnp.float32)
    y1 = _unpack_rows(y1_ref[...]).astype(jnp.float32)
    out_ref[...] = s + tw[:, 0:1] * y0 + tw[:, 1:2] * y1


def _combine(xbf, sfc1_w, sfc1_b, sfc2_w, sfc2_b, ypair, top_w):
    n = xbf.shape[0]
    nblk = n // _TB
    return pl.pallas_call(
        _combine_body,
        grid=(nblk,),
        in_specs=[
            pl.BlockSpec((_TB, _DH), lambda b: (b, 0)),
            pl.BlockSpec((_INTER, _D), lambda b: (0, 0)),
            pl.BlockSpec((1, _INTER), lambda b: (0, 0)),
            pl.BlockSpec((_D, _INTER), lambda b: (0, 0)),
            pl.BlockSpec((1, _D), lambda b: (0, 0)),
            pl.BlockSpec((_TB, _DH), lambda b: (b, 0)),
            pl.BlockSpec((_TB, _DH), lambda b, _nb=nblk: (b + _nb, 0)),
            pl.BlockSpec((_TB, _TOPK), lambda b: (b, 0)),
        ],
        out_specs=pl.BlockSpec((_TB, _D), lambda b: (b, 0)),
        out_shape=jax.ShapeDtypeStruct((n, _D), jnp.float32),
    )(xbf, sfc1_w, sfc1_b, sfc2_w, sfc2_b, ypair, ypair, top_w)


def kernel(x, gate_w, fc1_w, fc1_b, fc2_w, fc2_b, sfc1_w, sfc1_b, sfc2_w, sfc2_b):
    bb, hh, ww, dm = x.shape
    n = bb * hh * ww
    p = n * _TOPK
    xf = x.reshape(n, dm)

    row = jnp.arange(_GATE_TB, dtype=jnp.int32)
    tril = (row[:, None] > row[None, :]).astype(jnp.float32)
    top_idx, top_w, rank2, counts2, xbf = _gate(xf, gate_w, tril)

    counts = counts2.reshape(-1)
    blocks_e = (counts + _BLK - 1) // _BLK
    first_blk = jnp.cumsum(blocks_e) - blocks_e
    pad_offs = _BLK * first_blk
    block_expert = jnp.repeat(jnp.arange(_E, dtype=jnp.int32), blocks_e,
                              total_repeat_length=_NB)
    block_ord = jnp.arange(_NB, dtype=jnp.int32) - first_blk[block_expert]
    block_len = jnp.clip(counts[block_expert] - block_ord * _BLK, 0, _BLK)
    meta = jnp.concatenate([block_expert, block_len]).astype(jnp.int32)
    pp2 = _pp_compute(top_idx, rank2, pad_offs.reshape(1, _E))
    pp = pp2.T.reshape(-1)

    xg = _sc_dispatch(xbf, pp, _NB * _BLK)

    yg = _grouped_gemm(xg, fc1_w, fc1_b.reshape(_E, 1, _INTER),
                       fc2_w, fc2_b.reshape(_E, 1, _D), meta)

    ypair = _sc_undispatch(yg, pp)

    out = _combine(xbf, sfc1_w, sfc1_b.reshape(1, _INTER),
                   sfc2_w, sfc2_b.reshape(1, _D), ypair, top_w)
    return out.reshape(bb, hh, ww, dm)
